# Initial kernel scaffold; baseline (speedup 1.0000x reference)
import jax
import jax.numpy as jnp
from jax.experimental import pallas as pl


def kernel(x, params, src0, dst0, w0, src1, dst1, w1, src2, dst2, w2):
    raise NotImplementedError("write your pallas kernel here")


# speedup vs baseline: 9.3669x; 9.3669x over previous
import functools

import numpy as np
import jax
import jax.numpy as jnp
from jax import lax
from jax.experimental import pallas as pl
from jax.experimental.pallas import tpu as pltpu
from jax.experimental.pallas import tpu_sc as plsc

_NX = [64, 32, 16]
_NY = [64, 32, 16]
_NO = 6
_B = 8
_K = 4
_EPS = 1e-5
_NWORK = 32


@functools.lru_cache(maxsize=None)
def _level_consts(lvl):
    nx_, ny_ = _NX[lvl], _NY[lvl]
    ax, ay = np.arange(nx_), np.arange(ny_)
    degx = np.where((ax == 0) | (ax == nx_ - 1), 1.0, 2.0)
    degy = np.where((ay == 0) | (ay == ny_ - 1), 1.0, 2.0)
    deg = (degy[:, None] + degx[None, :] + 2.0).astype(np.float32)
    d = 1.0 / np.sqrt(deg)
    e = 1.0 / deg
    ml = (ax > 0).astype(np.float32)[None, :]
    mr = (ax < nx_ - 1).astype(np.float32)[None, :]
    mu = (ay > 0).astype(np.float32)[:, None]
    md = (ay < ny_ - 1).astype(np.float32)[:, None]
    c = {
        "cl": (-e * ml).ravel(),
        "cr": (-e * mr).ravel(),
        "cu": (-e * mu).ravel(),
        "cd": (-e * md).ravel(),
        "ne": (-e).ravel(),
        "d": d.ravel(),
    }
    c = {k: jnp.asarray(v) for k, v in c.items()}
    c["dfull"] = jnp.asarray(np.tile(d.ravel(), _NO))
    c["dinv"] = jnp.asarray(np.tile(np.sqrt(deg).ravel(), _NO))
    return c


@functools.lru_cache(maxsize=None)
def _sc_cheb(lvl, BC, Cin):
    nx_, ny_ = _NX[lvl], _NY[lvl]
    SL = ny_ * nx_
    V = _NO * SL
    PAD = nx_
    ppw = -(-BC // _NWORK)
    ncols = SL // 16
    mesh = plsc.VectorSubcoreMesh(core_axis_name="c", subcore_axis_name="s",
                                  num_cores=2, num_subcores=16)
    fdt = jnp.float32

    def body(h_hbm, ab_hbm, cl_h, cr_h, cu_h, cd_h, ne_h, d_h,
             z1_hbm, z2_hbm, z3_hbm,
             zA, zB, zC, clv, crv, cuv, cdv, nev, dvv, abv,
             s1, s2, s3):
        wid = lax.axis_index("s") * 2 + lax.axis_index("c")
        pltpu.sync_copy(cl_h, clv)
        pltpu.sync_copy(cr_h, crv)
        pltpu.sync_copy(cu_h, cuv)
        pltpu.sync_copy(cd_h, cdv)
        pltpu.sync_copy(ne_h, nev)
        pltpu.sync_copy(d_h, dvv)
        zeros16 = jnp.zeros((16,), fdt)
        for buf in (zA, zB, zC):
            for i in range(PAD // 16):
                buf[pl.ds(i * 16, 16)] = zeros16
                buf[pl.ds(PAD + V + i * 16, 16)] = zeros16

        def s_apply(zin, zprev, zout, first):
            def col(colid, _):
                p = colid * 16
                clv16 = clv[pl.ds(p, 16)]
                crv16 = crv[pl.ds(p, 16)]
                cuv16 = cuv[pl.ds(p, 16)]
                cdv16 = cdv[pl.ds(p, 16)]
                nev16 = nev[pl.ds(p, 16)]
                zo = [zin[pl.ds(PAD + o * SL + p, 16)] for o in range(_NO)]
                for o in range(_NO):
                    base = PAD + o * SL + p
                    lft = zin[pl.ds(base - 1, 16)]
                    rgt = zin[pl.ds(base + 1, 16)]
                    up = zin[pl.ds(base - nx_, 16)]
                    dn = zin[pl.ds(base + nx_, 16)]
                    osum = zo[(o - 1) % _NO] + zo[(o + 1) % _NO]
                    t = (clv16 * lft + crv16 * rgt + cuv16 * up
                         + cdv16 * dn + nev16 * osum)
                    if first:
                        zout[pl.ds(base, 16)] = t
                    else:
                        zout[pl.ds(base, 16)] = (t + t) - zprev[pl.ds(base, 16)]
                return 0

            lax.fori_loop(0, ncols, col, 0)

        def one_pair(idx):
            pltpu.sync_copy(ab_hbm.at[pl.ds(idx * 32, 32)], abv)
            Av = abv[pl.ds(0, 16)]
            Bv = abv[pl.ds(16, 16)]
            pltpu.sync_copy(h_hbm.at[pl.ds(idx * V, V)], zA.at[pl.ds(PAD, V)])

            def z0col(colid, _):
                p = colid * 16
                dv16 = dvv[pl.ds(p, 16)]
                for o in range(_NO):
                    base = PAD + o * SL + p
                    zA[pl.ds(base, 16)] = (zA[pl.ds(base, 16)] * Av + Bv) * dv16
                return 0

            lax.fori_loop(0, ncols, z0col, 0)
            dst = pl.ds(idx * V, V)
            s_apply(zA, None, zB, True)
            d1 = pltpu.async_copy(zB.at[pl.ds(PAD, V)], z1_hbm.at[dst], s1)
            s_apply(zB, zA, zC, False)
            d2 = pltpu.async_copy(zC.at[pl.ds(PAD, V)], z2_hbm.at[dst], s2)
            s_apply(zC, zB, zA, False)
            d3 = pltpu.async_copy(zA.at[pl.ds(PAD, V)], z3_hbm.at[dst], s3)
            d1.wait()
            d2.wait()
            d3.wait()

        for pair in range(ppw):
            idx = wid * ppw + pair
            if BC % _NWORK == 0:
                one_pair(idx)
            else:
                @pl.when(idx < BC)
                def _():
                    one_pair(idx)

    out = [jax.ShapeDtypeStruct((BC * V,), fdt)] * 3
    scratch = (
        [pltpu.VMEM((V + 2 * PAD,), fdt)] * 3
        + [pltpu.VMEM((SL,), fdt)] * 6
        + [pltpu.VMEM((32,), fdt)]
        + [pltpu.SemaphoreType.DMA] * 3
    )
    return pl.kernel(body, out_type=out, mesh=mesh, scratch_types=scratch,
                     name=f"sc_cheb_l{lvl}_bc{BC}")


def _tc_stats0(x, g1, be1):
    V = x.shape[2]

    def body(x_ref, g_ref, b_ref, ab_ref):
        xb = x_ref[...]
        n = float(x.shape[0] * V)
        s1 = jnp.sum(xb) / n
        s2 = jnp.sum(xb * xb) / n
        var = s2 - s1 * s1
        a = g_ref[...] * lax.rsqrt(var + _EPS)
        b = b_ref[...] - s1 * a
        ab_ref[...] = jnp.concatenate([a[None, :], b[None, :]], axis=0)

    return pl.pallas_call(
        body,
        in_specs=[pl.BlockSpec(x.shape, lambda: (0, 0, 0)),
                  pl.BlockSpec((16,), lambda: (0,)),
                  pl.BlockSpec((16,), lambda: (0,))],
        out_specs=pl.BlockSpec((2, 16), lambda: (0, 0)),
        out_shape=jax.ShapeDtypeStruct((2, 16), jnp.float32),
        name="tc_stats0",
    )(x, g1, be1)


def _tc_conv(lvl, Cin, Cout, mode, h, ab, z1, z2, z3, W, bias, gn=None, bn=None):
    nx_, ny_ = _NX[lvl], _NY[lvl]
    V = _NO * ny_ * nx_
    R2 = _NO * ny_ // 2
    cst = _level_consts(lvl)
    dfull, dinv = cst["dfull"], cst["dinv"]

    _DN = (((0,), (0,)), ((), ()))

    def compute_h(refs):
        (h_ref, ab_ref, z1_ref, z2_ref, z3_ref, W_ref, b_ref, df_ref,
         di_ref) = refs
        A = ab_ref[0, :Cin][:, None]
        Bc = ab_ref[1, :Cin][:, None]
        z0 = (h_ref[0] * A + Bc) * df_ref[...][None, :]
        acc = lax.dot_general(W_ref[0], z0, _DN, preferred_element_type=jnp.float32)
        acc += lax.dot_general(W_ref[1], z1_ref[0], _DN, preferred_element_type=jnp.float32)
        acc += lax.dot_general(W_ref[2], z2_ref[0], _DN, preferred_element_type=jnp.float32)
        acc += lax.dot_general(W_ref[3], z3_ref[0], _DN, preferred_element_type=jnp.float32)
        return acc * di_ref[...][None, :] + b_ref[...][:, None]

    if mode == "head":
        def body(h_ref, ab_ref, z1_ref, z2_ref, z3_ref, W_ref, b_ref,
                 df_ref, di_ref, out_ref):
            A = ab_ref[0, :Cin][:, None]
            Bc = ab_ref[1, :Cin][:, None]
            dn = (((0,), (0,)), ((), ()))
            rows = []
            for bb in range(_B):
                z0 = (h_ref[bb] * A + Bc) * df_ref[...][None, :]
                acc = lax.dot_general(W_ref[0], z0, dn, preferred_element_type=jnp.float32)
                acc += lax.dot_general(W_ref[1], z1_ref[bb], dn, preferred_element_type=jnp.float32)
                acc += lax.dot_general(W_ref[2], z2_ref[bb], dn, preferred_element_type=jnp.float32)
                acc += lax.dot_general(W_ref[3], z3_ref[bb], dn, preferred_element_type=jnp.float32)
                hout = acc * di_ref[...][None, :] + b_ref[...][:, None]
                m = jnp.max(hout, axis=1)
                mm = jnp.max(m)
                rows.append((m - (mm + jnp.log(jnp.sum(jnp.exp(m - mm)))))[None, :])
            out_ref[...] = jnp.concatenate(rows, axis=0)

        full = lambda shape: pl.BlockSpec(shape, lambda: tuple(0 for _ in shape))
        return pl.pallas_call(
            body,
            in_specs=[full((_B, Cin, V)), full((2, 16)), full((_B, Cin, V)),
                      full((_B, Cin, V)), full((_B, Cin, V)),
                      full((_K, Cin, Cout)), full((Cout,)), full((V,)),
                      full((V,))],
            out_specs=full((_B, 10)),
            out_shape=jax.ShapeDtypeStruct((_B, 10), jnp.float32),
            name=f"tc_head_l{lvl}",
        )(h, ab, z1, z2, z3, W, bias, dfull, dinv)

    pool = mode == "pool"
    if pool:
        E0 = jnp.asarray(np.eye(nx_, dtype=np.float32)[:, 0::2])
        E1 = jnp.asarray(np.eye(nx_, dtype=np.float32)[:, 1::2])
        out_hshape = (_B, Cout, R2, nx_ // 2)
        nstat = _B * (V // 4)
    else:
        out_hshape = (_B, Cout, V)
        nstat = _B * V

    def body(*refs):
        if pool:
            (h_ref, ab_ref, z1_ref, z2_ref, z3_ref, W_ref, b_ref, df_ref,
             di_ref, gn_ref, bn_ref, e0_ref, e1_ref, out_ref, abo_ref,
             s_ref) = refs
        else:
            (h_ref, ab_ref, z1_ref, z2_ref, z3_ref, W_ref, b_ref, df_ref,
             di_ref, gn_ref, bn_ref, out_ref, abo_ref, s_ref) = refs
        b = pl.program_id(0)
        hout = compute_h(refs[:9])
        if pool:
            hr = hout.reshape(Cout, R2, 2, nx_)
            ym = jnp.max(hr, axis=2)
            ev = lax.dot_general(ym, e0_ref[...], (((2,), (0,)), ((), ())),
                                 preferred_element_type=jnp.float32)
            od = lax.dot_general(ym, e1_ref[...], (((2,), (0,)), ((), ())),
                                 preferred_element_type=jnp.float32)
            hout = jnp.maximum(ev, od)
            out_ref[0] = hout
            red_axes = (1, 2)
        else:
            out_ref[0] = hout
            red_axes = (1,)

        @pl.when(b == 0)
        def _():
            s_ref[...] = jnp.zeros_like(s_ref)

        s1 = jnp.sum(hout, axis=red_axes)
        s2 = jnp.sum(hout * hout, axis=red_axes)
        s_ref[0, :] += s1
        s_ref[1, :] += s2

        @pl.when(b == _B - 1)
        def _():
            mean = s_ref[0, :] / float(nstat)
            var = s_ref[1, :] / float(nstat) - mean * mean
            a = gn_ref[...] * lax.rsqrt(var + _EPS)
            bc = bn_ref[...] - mean * a
            abo_ref[...] = jnp.concatenate([a[None, :], bc[None, :]], axis=0)

    zspec = pl.BlockSpec((1, Cin, V), lambda b: (b, 0, 0))
    cspec = lambda shape: pl.BlockSpec(shape, lambda b: tuple(0 for _ in shape))
    in_specs = [zspec, cspec((2, 16)), zspec, zspec, zspec,
                cspec((_K, Cin, Cout)), cspec((Cout,)), cspec((V,)),
                cspec((V,)), cspec((16,)), cspec((16,))]
    args = [h, ab, z1, z2, z3, W, bias, dfull, dinv, gn, bn]
    if pool:
        in_specs += [cspec((nx_, nx_ // 2)), cspec((nx_, nx_ // 2))]
        args += [E0, E1]
        out_specs = [pl.BlockSpec((1, Cout, R2, nx_ // 2), lambda b: (b, 0, 0, 0)),
                     cspec((2, 16))]
    else:
        out_specs = [pl.BlockSpec((1, Cout, V), lambda b: (b, 0, 0)),
                     cspec((2, 16))]
    return pl.pallas_call(
        body,
        grid=(_B,),
        in_specs=in_specs,
        out_specs=out_specs,
        out_shape=[jax.ShapeDtypeStruct(out_hshape, jnp.float32),
                   jax.ShapeDtypeStruct((2, 16), jnp.float32)],
        scratch_shapes=[pltpu.VMEM((2, 16), jnp.float32)],
        name=f"tc_conv_l{lvl}_{mode}",
    )(*args)


def _pad16(v):
    n = v.shape[0]
    if n == 16:
        return v
    return jnp.concatenate([v, jnp.zeros((16 - n,), v.dtype)])


def kernel(x, params, src0, dst0, w0, src1, dst1, w1, src2, dst2, w2):
    p = params
    Vs = [_NO * _NY[i] * _NX[i] for i in range(3)]

    def sc(lvl, h2d, ab, BC, Cin):
        cst = _level_consts(lvl)
        fn = _sc_cheb(lvl, BC, Cin)
        arow = jnp.tile(ab[0, :Cin], _B)[:, None]
        brow = jnp.tile(ab[1, :Cin], _B)[:, None]
        abrow = jnp.concatenate([jnp.broadcast_to(arow, (BC, 16)),
                                 jnp.broadcast_to(brow, (BC, 16))],
                                axis=1).reshape(-1)
        return fn(h2d.reshape(-1), abrow, cst["cl"], cst["cr"], cst["cu"],
                  cst["cd"], cst["ne"], cst["d"])

    ab0 = _tc_stats0(x, _pad16(p["g1"]), _pad16(p["be1"]))
    z1, z2, z3 = sc(0, x.reshape(_B, Vs[0]), ab0, _B, 1)
    r = lambda z, C, V: z.reshape(_B, C, V)
    h1, ab1 = _tc_conv(0, 1, 16, "plain", x, ab0,
                       r(z1, 1, Vs[0]), r(z2, 1, Vs[0]), r(z3, 1, Vs[0]),
                       p["W1"], p["b1"], _pad16(p["g2"]), _pad16(p["be2"]))
    z1, z2, z3 = sc(0, h1.reshape(_B * 16, Vs[0]), ab1, _B * 16, 16)
    h2, ab2 = _tc_conv(0, 16, 16, "pool", h1, ab1,
                       r(z1, 16, Vs[0]), r(z2, 16, Vs[0]), r(z3, 16, Vs[0]),
                       p["W2"], p["b2"], _pad16(p["g3"]), _pad16(p["be3"]))
    h2 = h2.reshape(_B, 16, Vs[1])
    z1, z2, z3 = sc(1, h2.reshape(_B * 16, Vs[1]), ab2, _B * 16, 16)
    h3, ab3 = _tc_conv(1, 16, 16, "plain", h2, ab2,
                       r(z1, 16, Vs[1]), r(z2, 16, Vs[1]), r(z3, 16, Vs[1]),
                       p["W3"], p["b3"], _pad16(p["g4"]), _pad16(p["be4"]))
    z1, z2, z3 = sc(1, h3.reshape(_B * 16, Vs[1]), ab3, _B * 16, 16)
    h4, ab4 = _tc_conv(1, 16, 16, "pool", h3, ab3,
                       r(z1, 16, Vs[1]), r(z2, 16, Vs[1]), r(z3, 16, Vs[1]),
                       p["W4"], p["b4"], _pad16(p["g5"]), _pad16(p["be5"]))
    h4 = h4.reshape(_B, 16, Vs[2])
    z1, z2, z3 = sc(2, h4.reshape(_B * 16, Vs[2]), ab4, _B * 16, 16)
    h5, ab5 = _tc_conv(2, 16, 16, "plain", h4, ab4,
                       r(z1, 16, Vs[2]), r(z2, 16, Vs[2]), r(z3, 16, Vs[2]),
                       p["W5"], p["b5"], _pad16(p["g6"]), _pad16(p["be6"]))
    z1, z2, z3 = sc(2, h5.reshape(_B * 16, Vs[2]), ab5, _B * 16, 16)
    out = _tc_conv(2, 16, 10, "head", h5, ab5,
                   r(z1, 16, Vs[2]), r(z2, 16, Vs[2]), r(z3, 16, Vs[2]),
                   p["W6"], p["b6"])
    return out
